# Initial kernel scaffold; baseline (speedup 1.0000x reference)
import jax
import jax.numpy as jnp
from jax.experimental import pallas as pl


def kernel(x, edge_index, W1, b1, W2, b2):
    raise NotImplementedError("write your pallas kernel here")


# speedup vs baseline: 17.8004x; 17.8004x over previous
import functools

import jax
import jax.numpy as jnp
from jax import lax
from jax.experimental import pallas as pl
from jax.experimental.pallas import tpu as pltpu
from jax.experimental.pallas import tpu_sc as plsc

N = 10000
F = 128
H = 256
C = 64
E = 320000

NC = 2
NS = 16
NW = NC * NS
EB = 128
NB2 = 80
NB1 = 160
E_PAD = NW * NB2 * EB
N_ACC = 10112
RPT = N_ACC // NS
HD = 64

_mesh = plsc.VectorSubcoreMesh(core_axis_name="c", subcore_axis_name="s")
_sc_params = pltpu.CompilerParams(needs_layout_passes=False,
                                  use_tc_tiling_on_sc=False)


@functools.partial(
    pl.kernel,
    out_type=jax.ShapeDtypeStruct((NW, N_ACC), jnp.float32),
    mesh=_mesh,
    compiler_params=_sc_params,
    scratch_types=[
        pltpu.VMEM((NB2 * EB,), jnp.int32),
        pltpu.VMEM((N_ACC,), jnp.float32),
    ],
)
def _deg_kernel(dst_hbm, out_hbm, dst_v, deg_v):
    c = lax.axis_index("c")
    s = lax.axis_index("s")
    wid = c * NS + s
    pltpu.sync_copy(dst_hbm.at[wid], dst_v)

    zero16 = jnp.zeros((16,), jnp.float32)

    def zbody(i, _):
        deg_v[pl.ds(i * 16, 16)] = zero16
        return 0

    lax.fori_loop(0, N_ACC // 16, zbody, 0)

    one16 = jnp.ones((16,), jnp.float32)

    def body(i, _):
        idx = dst_v[pl.ds(i * 16, 16)]
        plsc.addupdate_scatter(deg_v, [idx], one16)
        return 0

    lax.fori_loop(0, (NB2 * EB) // 16, body, 0)
    pltpu.sync_copy(deg_v, out_hbm.at[wid])


def _make_agg_kernel(nb, feature_split):
    tshape = (NC, N, HD) if feature_split else (N, HD)

    @functools.partial(
        pl.kernel,
        out_type=jax.ShapeDtypeStruct((NC, N_ACC, HD), jnp.float32),
        mesh=_mesh,
        compiler_params=_sc_params,
        scratch_types=[
            pltpu.VMEM((nb, EB), jnp.int32),
            pltpu.VMEM((nb, EB), jnp.int32),
            pltpu.VMEM((EB, HD), jnp.float32),
            pltpu.VMEM((EB, HD), jnp.float32),
            pltpu.VMEM_SHARED((N_ACC, HD), jnp.float32),
            pltpu.SemaphoreType.DMA,
            pltpu.SemaphoreType.DMA,
        ],
    )
    def _agg(table_hbm, src_hbm, dst_hbm, zeros_hbm, out_hbm,
             src_v, dst_v, buf0, buf1, acc_sh, sem0, sem1):
        c = lax.axis_index("c")
        s = lax.axis_index("s")
        slab = s if feature_split else c * NS + s
        table = table_hbm.at[c] if feature_split else table_hbm
        pltpu.sync_copy(src_hbm.at[slab], src_v)
        pltpu.sync_copy(dst_hbm.at[slab], dst_v)
        pltpu.sync_copy(zeros_hbm.at[pl.ds(s * RPT, RPT)],
                        acc_sh.at[pl.ds(s * RPT, RPT)])
        plsc.subcore_barrier()

        pltpu.async_copy(table.at[src_v.at[0]], buf0, sem0)

        def body(i, _):
            b0 = 2 * i
            b1 = b0 + 1
            pltpu.async_copy(table.at[src_v.at[b1]], buf1, sem1)
            pltpu.make_async_copy(table.at[src_v.at[b0]], buf0, sem0).wait()
            pltpu.sync_copy(buf0, acc_sh.at[dst_v.at[b0]], add=True)

            @pl.when(b0 + 2 < nb)
            def _():
                pltpu.async_copy(table.at[src_v.at[b0 + 2]], buf0, sem0)

            pltpu.make_async_copy(table.at[src_v.at[b1]], buf1, sem1).wait()
            pltpu.sync_copy(buf1, acc_sh.at[dst_v.at[b1]], add=True)
            return 0

        lax.fori_loop(0, nb // 2, body, 0)
        plsc.subcore_barrier()
        pltpu.sync_copy(acc_sh.at[pl.ds(s * RPT, RPT)],
                        out_hbm.at[c, pl.ds(s * RPT, RPT)])

    return _agg


_agg1 = _make_agg_kernel(NB1, True)
_agg2 = _make_agg_kernel(NB2, False)


def _dinv_from_parts(degp):
    deg = jnp.sum(degp, axis=0)[:N] + 1.0
    return lax.rsqrt(deg)[:, None]


def _tc1_body(degp_ref, x_ref, y1_ref):
    dinv = _dinv_from_parts(degp_ref[...])
    y1_ref[0] = x_ref[:, :HD] * dinv
    y1_ref[1] = x_ref[:, HD:] * dinv


def _tc1(deg_parts, x):
    return pl.pallas_call(
        _tc1_body,
        out_shape=jax.ShapeDtypeStruct((NC, N, HD), jnp.float32),
    )(deg_parts, x)


def _tc2_body(acc_ref, y1_ref, degp_ref, w1_ref, b1_ref, w2_ref, b2_ref, y2_ref):
    dinv = _dinv_from_parts(degp_ref[...])
    z = jnp.concatenate(
        [acc_ref[0, :N, :] + y1_ref[0], acc_ref[1, :N, :] + y1_ref[1]],
        axis=1) * dinv
    h = jnp.dot(z, w1_ref[...], preferred_element_type=jnp.float32) + b1_ref[...]
    h = jnp.maximum(h, 0.0)
    y2 = jnp.dot(h, w2_ref[...], preferred_element_type=jnp.float32)
    y2_ref[...] = y2 * dinv


def _tc2(acc1, y1_two, deg_parts, W1, b1, W2, b2):
    return pl.pallas_call(
        _tc2_body,
        out_shape=jax.ShapeDtypeStruct((N, C), jnp.float32),
    )(acc1, y1_two, deg_parts, W1, b1, W2, b2)


def _tc3_body(acc_ref, y2_ref, degp_ref, b2_ref, out_ref):
    dinv = _dinv_from_parts(degp_ref[...])
    out_ref[...] = (acc_ref[0, :N, :] + acc_ref[1, :N, :]
                    + y2_ref[...]) * dinv + b2_ref[...]


def _tc3(acc2, y2, deg_parts, b2):
    return pl.pallas_call(
        _tc3_body,
        out_shape=jax.ShapeDtypeStruct((N, C), jnp.float32),
    )(acc2, y2, deg_parts, b2)


def kernel(x, edge_index, W1, b1, W2, b2):
    ei = edge_index.astype(jnp.int32)
    src, dst = ei[0], ei[1]
    pad = E_PAD - E
    src_pad = jnp.concatenate([src, jnp.zeros((pad,), jnp.int32)])
    dst_pad = jnp.concatenate([dst, jnp.full((pad,), N, jnp.int32)])
    src32 = src_pad.reshape(NW, NB2, EB)
    dst32 = dst_pad.reshape(NW, NB2, EB)
    src16 = src_pad.reshape(NS, NB1, EB)
    dst16 = dst_pad.reshape(NS, NB1, EB)
    dst_flat = dst_pad.reshape(NW, NB2 * EB)

    zeros_hd = jnp.zeros((N_ACC, HD), jnp.float32)

    deg_parts = _deg_kernel(dst_flat)
    y1_two = _tc1(deg_parts, x)
    acc1 = _agg1(y1_two, src16, dst16, zeros_hd)
    y2 = _tc2(acc1, y1_two, deg_parts,
              W1, b1.reshape(1, H), W2, b2.reshape(1, C))
    acc2 = _agg2(y2, src32, dst32, zeros_hd)
    return _tc3(acc2, y2, deg_parts, b2.reshape(1, C))
